# baseline (speedup 1.0000x reference)
import functools

import jax
import jax.numpy as jnp
from jax import lax
from jax.experimental import pallas as pl
from jax.experimental.pallas import tpu as pltpu
from jax.experimental.pallas import tpu_sc as plsc

_TOP_P = 0.8
_E = 64
_T_BLK = 1024
_N_TOK = 32768
_NW = 32
_TOK_PER_W = _N_TOK // _NW
_SLAB = 512
_LANES = 16


def _probs_kernel(x_ref, w_ref, o_ref):
    x = x_ref[...]
    w = w_ref[...]
    logits_t = jax.lax.dot_general(
        w, x, (((1,), (1,)), ((), ())),
        preferred_element_type=jnp.float32,
    )
    m = jnp.max(logits_t, axis=0, keepdims=True)
    ex = jnp.exp(logits_t - m)
    o_ref[...] = ex / jnp.sum(ex, axis=0, keepdims=True)


def _tc_probs(routing_inputs, W):
    n_tok, hidden = routing_inputs.shape
    return pl.pallas_call(
        _probs_kernel,
        grid=(n_tok // _T_BLK,),
        in_specs=[
            pl.BlockSpec((_T_BLK, hidden), lambda i: (i, 0)),
            pl.BlockSpec((_E, hidden), lambda i: (0, 0)),
        ],
        out_specs=pl.BlockSpec((_E, _T_BLK), lambda i: (0, i)),
        out_shape=jax.ShapeDtypeStruct((_E, n_tok), jnp.float32),
    )(routing_inputs, W)


def _sc_gate_body(probs_hbm, out_hbm, p_v, o_v, sem):
    wid = lax.axis_index("s") * 2 + lax.axis_index("c")
    lane = lax.iota(jnp.int32, _LANES)

    for slab in range(_TOK_PER_W // _SLAB):
        base = wid * _TOK_PER_W + slab * _SLAB
        pltpu.sync_copy(probs_hbm.at[:, pl.ds(base, _SLAB)], p_v)

        def group_body(g, carry):
            off = g * _LANES

            def gather_row(j):
                return p_v[j, pl.ds(off, _LANES)]

            def bisect(_, lohi):
                lo, hi = lohi
                mid_bits = (lo + hi) >> 1
                mid = lax.bitcast_convert_type(mid_bits, jnp.float32)
                s = jnp.zeros((_LANES,), jnp.float32)
                for j in range(_E):
                    pj = gather_row(j)
                    s = s + jnp.where(pj > mid, pj, 0.0)
                gt = s > _TOP_P
                return (jnp.where(gt, mid_bits, lo),
                        jnp.where(gt, hi, mid_bits))

            lo0 = jnp.zeros((_LANES,), jnp.int32)
            hi0 = jnp.full((_LANES,), 0x3F800000, jnp.int32)
            lo, hi = lax.fori_loop(0, 31, bisect, (lo0, hi0))
            theta = lax.bitcast_convert_type(hi, jnp.float32)

            s_gt = jnp.zeros((_LANES,), jnp.float32)
            for j in range(_E):
                pj = gather_row(j)
                s_gt = s_gt + jnp.where(pj > theta, pj, 0.0)
            m_keep = jnp.minimum((_TOP_P - s_gt) / theta, 64.0
                                 ).astype(jnp.int32) + 1

            eq_rank = jnp.zeros((_LANES,), jnp.int32)
            for j in range(_E):
                pj = gather_row(j)
                eq = pj == theta
                kept = (pj > theta) | (eq & (eq_rank < m_keep))
                o_v[j, pl.ds(off, _LANES)] = jnp.where(kept, 1.0, 0.0)
                eq_rank = eq_rank + jnp.where(eq, 1, 0)
            return carry

        lax.fori_loop(0, _SLAB // _LANES, group_body, 0)
        pltpu.sync_copy(o_v, out_hbm.at[:, pl.ds(base, _SLAB)])


def _sc_gate(probs_t):
    mesh = plsc.VectorSubcoreMesh(core_axis_name="c", subcore_axis_name="s")
    gate = pl.kernel(
        _sc_gate_body, mesh=mesh,
        out_type=jax.ShapeDtypeStruct((_E, _N_TOK), jnp.float32),
        scratch_types=[
            pltpu.VMEM((_E, _SLAB), jnp.float32),
            pltpu.VMEM((_E, _SLAB), jnp.float32),
            pltpu.SemaphoreType.DMA,
        ],
        compiler_params=pltpu.CompilerParams(use_tc_tiling_on_sc=False),
    )
    return gate(probs_t)


def kernel(routing_inputs, W):
    probs_t = _tc_probs(routing_inputs, W)
    mask_t = _sc_gate(probs_t)
    return jnp.transpose(mask_t)


# speedup vs baseline: 1.5362x; 1.5362x over previous
import jax
import jax.numpy as jnp
from jax.experimental import pallas as pl
from jax.experimental.pallas import tpu as pltpu

_TOP_P = 0.8
_E = 64
_T_BLK = 1024


def _gate_kernel(x_ref, w_ref, o_ref, acc_ref):
    x = x_ref[...]
    w = w_ref[...]
    partial = jax.lax.dot_general(
        w, x, (((1,), (1,)), ((), ())),
        preferred_element_type=jnp.float32,
    )
    k = pl.program_id(1)
    nk = pl.num_programs(1)

    @pl.when(k == 0)
    def _():
        acc_ref[...] = partial

    @pl.when(k != 0)
    def _():
        acc_ref[...] = acc_ref[...] + partial

    @pl.when(k == nk - 1)
    def _():
        _finish(acc_ref[...], o_ref)


def _finish(logits_t, o_ref):
    m = jnp.max(logits_t, axis=0, keepdims=True)
    ex = jnp.exp(logits_t - m)
    probs = ex / jnp.sum(ex, axis=0, keepdims=True)

    row = jax.lax.broadcasted_iota(jnp.int32, probs.shape, 0)
    rows = []
    for e in range(_E):
        pe = probs[e:e + 1, :]
        above = (probs > pe) | ((probs == pe) & (row < e))
        s_e = jnp.sum(jnp.where(above, probs, 0.0), axis=0, keepdims=True)
        rows.append(s_e)
    s = jnp.concatenate(rows, axis=0)
    out_t = jnp.where(s <= _TOP_P, 1.0, 0.0)
    eye = (jax.lax.broadcasted_iota(jnp.int32, (_E, _E), 0)
           == jax.lax.broadcasted_iota(jnp.int32, (_E, _E), 1)
           ).astype(jnp.float32)
    o_ref[...] = jax.lax.dot_general(
        out_t, eye, (((0,), (0,)), ((), ())),
        preferred_element_type=jnp.float32,
    )


_N_K = 2


def kernel(routing_inputs, W):
    n_tok, hidden = routing_inputs.shape
    hk = hidden // _N_K
    return pl.pallas_call(
        _gate_kernel,
        grid=(n_tok // _T_BLK, _N_K),
        in_specs=[
            pl.BlockSpec((_T_BLK, hk), lambda i, k: (i, k)),
            pl.BlockSpec((_E, hk), lambda i, k: (0, k)),
        ],
        out_specs=pl.BlockSpec((_T_BLK, _E), lambda i, k: (i, 0)),
        out_shape=jax.ShapeDtypeStruct((n_tok, _E), jnp.float32),
        scratch_shapes=[pltpu.VMEM((_E, _T_BLK), jnp.float32)],
        compiler_params=pltpu.CompilerParams(
            dimension_semantics=("parallel", "arbitrary"),
        ),
    )(routing_inputs, W)


# speedup vs baseline: 2.0883x; 1.3594x over previous
import jax
import jax.numpy as jnp
from jax.experimental import pallas as pl
from jax.experimental.pallas import tpu as pltpu

_TOP_P = 0.8
_E = 64
_T_BLK = 1024


def _gate_kernel(x_ref, w_ref, o_ref):
    x = x_ref[...]
    w = w_ref[...]
    logits_t = jax.lax.dot_general(
        w, x, (((1,), (1,)), ((), ())),
        preferred_element_type=jnp.float32,
    )
    m = jnp.max(logits_t, axis=0, keepdims=True)
    ex = jnp.exp(logits_t - m)
    probs = ex / jnp.sum(ex, axis=0, keepdims=True)

    row = jax.lax.broadcasted_iota(jnp.int32, probs.shape, 0)
    rows = []
    for e in range(_E):
        pe = probs[e:e + 1, :]
        above = (probs > pe) | ((probs == pe) & (row < e))
        s_e = jnp.sum(jnp.where(above, probs, 0.0), axis=0, keepdims=True)
        rows.append(s_e)
    s = jnp.concatenate(rows, axis=0)
    out_t = jnp.where(s <= _TOP_P, 1.0, 0.0)
    eye = (jax.lax.broadcasted_iota(jnp.int32, (_E, _E), 0)
           == jax.lax.broadcasted_iota(jnp.int32, (_E, _E), 1)
           ).astype(jnp.float32)
    o_ref[...] = jax.lax.dot_general(
        out_t, eye, (((0,), (0,)), ((), ())),
        preferred_element_type=jnp.float32,
    )


def kernel(routing_inputs, W):
    n_tok, hidden = routing_inputs.shape
    return pl.pallas_call(
        _gate_kernel,
        grid=(n_tok // _T_BLK,),
        in_specs=[
            pl.BlockSpec((_T_BLK, hidden), lambda i: (i, 0)),
            pl.BlockSpec((_E, hidden), lambda i: (0, 0)),
        ],
        out_specs=pl.BlockSpec((_T_BLK, _E), lambda i: (i, 0)),
        out_shape=jax.ShapeDtypeStruct((n_tok, _E), jnp.float32),
        compiler_params=pltpu.CompilerParams(
            dimension_semantics=("parallel",),
        ),
    )(routing_inputs, W)
